# Initial kernel scaffold; baseline (speedup 1.0000x reference)
import jax
import jax.numpy as jnp
from jax.experimental import pallas as pl


def kernel(x, edge_index, batch, W1, b1, W2, b2, Wm, bm, Wt, bt):
    raise NotImplementedError("write your pallas kernel here")


# speedup vs baseline: 3.0722x; 3.0722x over previous
import functools

import jax
import jax.numpy as jnp
from jax import lax
from jax.experimental import pallas as pl
from jax.experimental.pallas import tpu as pltpu
from jax.experimental.pallas import tpu_sc as plsc

_N = 10000
_E = 320000
_D = 128
_G = 64
_NC = 2
_NS = 16
_NT = _NC * _NS
_EPT = _E // _NT
_CH = 80
_NCHK = _EPT // _CH
_NSUP = 5
_NSCH = _NCHK // _NSUP
_RB = 2000
_NRB = _N // _RB


def _mesh():
    return plsc.VectorSubcoreMesh(core_axis_name="c", subcore_axis_name="s")


def _sc_degree(dst3):
    @functools.partial(
        pl.kernel,
        out_type=jax.ShapeDtypeStruct((_NC * _N, 16), jnp.float32),
        mesh=_mesh(),
        scratch_types=[
            pltpu.VMEM((_NSCH, _CH), jnp.int32),
            pltpu.VMEM((_CH,), jnp.int32),
            pltpu.VMEM((_CH, 16), jnp.float32),
            pltpu.VMEM((640, 16), jnp.float32),
            pltpu.VMEM_SHARED((_N, 16), jnp.float32),
            pltpu.SemaphoreType.DMA,
        ],
    )
    def k(dst_hbm, out_hbm, idx_v, idx_c, ones_v, stage_v, acc, sem):
        c = lax.axis_index("c")
        s = lax.axis_index("s")
        tid = c * _NS + s

        def fill_ones(r, _):
            ones_v[r, :] = jnp.ones((16,), jnp.float32)
            return 0

        lax.fori_loop(0, _CH, fill_ones, 0)

        def fill_zero(r, _):
            stage_v[r, :] = jnp.zeros((16,), jnp.float32)
            return 0

        lax.fori_loop(0, 640, fill_zero, 0)

        for ss in range(_NS):
            @pl.when(s == ss)
            def _(ss=ss):
                pltpu.sync_copy(stage_v.at[pl.ds(0, 400)],
                                acc.at[pl.ds(ss * 400, 400)])
                pltpu.sync_copy(acc.at[pl.ds(ss * 400, 400)],
                                stage_v.at[pl.ds(0, 400)])

        pltpu.sync_copy(stage_v.at[pl.ds(0, 400)],
                        out_hbm.at[pl.ds(tid * 400, 400)])

    return k(dst3)


def _sc_aggregate(ph, src3, dst3):
    @functools.partial(
        pl.kernel,
        out_type=jax.ShapeDtypeStruct((_NC * _N, _D), jnp.float32),
        mesh=_mesh(),
        scratch_types=[
            pltpu.VMEM((_NSCH, _CH), jnp.int32),
            pltpu.VMEM((_NSCH, _CH), jnp.int32),
            pltpu.VMEM((_CH, _D), jnp.float32),
            pltpu.VMEM((80, _D), jnp.float32),
            pltpu.VMEM_SHARED((_N, _D), jnp.float32),
            pltpu.SemaphoreType.DMA,
        ],
    )
    def k(ph_hbm, src_hbm, dst_hbm, out_hbm, src_v, dst_v, rows_v, stage_v,
          acc, sem):
        c = lax.axis_index("c")
        s = lax.axis_index("s")
        tid = c * _NS + s

        def fill_zero(r, _):
            for j in range(_D // 16):
                stage_v[r, pl.ds(j * 16, 16)] = jnp.zeros((16,), jnp.float32)
            return 0

        lax.fori_loop(0, 80, fill_zero, 0)

        @pl.when(s < _NS - 1)
        def _():
            for r in range(8):
                pltpu.sync_copy(stage_v, acc.at[pl.ds(s * 640 + r * 80, 80)])

        @pl.when(s == _NS - 1)
        def _():
            for r in range(5):
                pltpu.sync_copy(stage_v, acc.at[pl.ds(9600 + r * 80, 80)])

        plsc.subcore_barrier()

        def inner(g, _):
            pltpu.async_copy(ph_hbm.at[src_v.at[g]], rows_v, sem).wait()
            pltpu.sync_copy(rows_v, acc.at[dst_v.at[g]], add=True)
            return 0

        def outer(u, _):
            pltpu.sync_copy(src_hbm.at[tid * _NSUP + u], src_v)
            pltpu.sync_copy(dst_hbm.at[tid * _NSUP + u], dst_v)
            lax.fori_loop(0, _NSCH, inner, 0)
            return 0

        lax.fori_loop(0, _NSUP, outer, 0)
        plsc.subcore_barrier()

        @pl.when(s < _NS - 1)
        def _():
            for r in range(8):
                off = s * 640 + r * 80
                pltpu.sync_copy(acc.at[pl.ds(off, 80)], stage_v)
                pltpu.sync_copy(stage_v, out_hbm.at[pl.ds(c * _N + off, 80)])

        @pl.when(s == _NS - 1)
        def _():
            for r in range(5):
                off = 9600 + r * 80
                pltpu.sync_copy(acc.at[pl.ds(off, 80)], stage_v)
                pltpu.sync_copy(stage_v, out_hbm.at[pl.ds(c * _N + off, 80)])

    return k(ph, src3, dst3)


def _tc1_body(x_ref, w_ref, da_ref, db_ref, ph_ref, dinv_ref):
    deg = da_ref[:, 0:1] + db_ref[:, 0:1] + 1.0
    dinv = lax.rsqrt(deg)
    dinv_ref[...] = dinv
    ph_ref[...] = jnp.dot(x_ref[...], w_ref[...],
                          preferred_element_type=jnp.float32) * dinv


def _tc1(x, W1, dega, degb):
    return pl.pallas_call(
        _tc1_body,
        grid=(_NRB,),
        in_specs=[
            pl.BlockSpec((_RB, _D), lambda i: (i, 0)),
            pl.BlockSpec((_D, _D), lambda i: (0, 0)),
            pl.BlockSpec((_RB, 16), lambda i: (i, 0)),
            pl.BlockSpec((_RB, 16), lambda i: (i, 0)),
        ],
        out_specs=[
            pl.BlockSpec((_RB, _D), lambda i: (i, 0)),
            pl.BlockSpec((_RB, 1), lambda i: (i, 0)),
        ],
        out_shape=[
            jax.ShapeDtypeStruct((_N, _D), jnp.float32),
            jax.ShapeDtypeStruct((_N, 1), jnp.float32),
        ],
    )(x, W1, dega, degb)


def _tc2_body(sa_ref, sb_ref, ph_ref, dinv_ref, b_ref, w_ref, out_ref):
    dinv = dinv_ref[...]
    t = sa_ref[...] + sb_ref[...] + ph_ref[...]
    h = jnp.maximum(t * dinv + b_ref[...], 0.0)
    out_ref[...] = jnp.dot(h, w_ref[...],
                           preferred_element_type=jnp.float32) * dinv


def _tc2(sa, sb, ph, dinv, b1, W2):
    return pl.pallas_call(
        _tc2_body,
        grid=(_NRB,),
        in_specs=[
            pl.BlockSpec((_RB, _D), lambda i: (i, 0)),
            pl.BlockSpec((_RB, _D), lambda i: (i, 0)),
            pl.BlockSpec((_RB, _D), lambda i: (i, 0)),
            pl.BlockSpec((_RB, 1), lambda i: (i, 0)),
            pl.BlockSpec((1, _D), lambda i: (0, 0)),
            pl.BlockSpec((_D, _D), lambda i: (0, 0)),
        ],
        out_specs=pl.BlockSpec((_RB, _D), lambda i: (i, 0)),
        out_shape=jax.ShapeDtypeStruct((_N, _D), jnp.float32),
    )(sa, sb, ph, dinv, b1, W2)


def _tc3_body(sa_ref, sb_ref, ph_ref, dinv_ref, b_ref, batch_ref,
              wm_ref, bm_ref, wt_ref, bt_ref, mem_ref, time_ref,
              pool_ref, cnt_ref):
    i = pl.program_id(0)
    t = sa_ref[...] + sb_ref[...] + ph_ref[...]
    h = jnp.maximum(t * dinv_ref[...] + b_ref[...], 0.0)
    gids = lax.broadcasted_iota(jnp.int32, (1, _G), 1)
    onehot = (batch_ref[...] == gids).astype(jnp.float32)
    dn = (((0,), (0,)), ((), ()))
    pp = lax.dot_general(onehot, h, dn,
                         preferred_element_type=jnp.float32)
    cc = lax.dot_general(onehot, jnp.ones((_RB, 1), jnp.float32), dn,
                         preferred_element_type=jnp.float32)

    @pl.when(i == 0)
    def _():
        pool_ref[...] = pp
        cnt_ref[...] = cc

    @pl.when(i > 0)
    def _():
        pool_ref[...] += pp
        cnt_ref[...] += cc

    @pl.when(i == _NRB - 1)
    def _():
        mean = pool_ref[...] / jnp.maximum(cnt_ref[...], 1.0)
        mem_ref[...] = jnp.dot(mean, wm_ref[...],
                               preferred_element_type=jnp.float32) + bm_ref[...]
        time_ref[...] = jnp.dot(mean, wt_ref[...],
                                preferred_element_type=jnp.float32) + bt_ref[...]


def _tc3(sa, sb, ph, dinv, b2, batch2, Wm, bm, Wt, bt):
    return pl.pallas_call(
        _tc3_body,
        grid=(_NRB,),
        in_specs=[
            pl.BlockSpec((_RB, _D), lambda i: (i, 0)),
            pl.BlockSpec((_RB, _D), lambda i: (i, 0)),
            pl.BlockSpec((_RB, _D), lambda i: (i, 0)),
            pl.BlockSpec((_RB, 1), lambda i: (i, 0)),
            pl.BlockSpec((1, _D), lambda i: (0, 0)),
            pl.BlockSpec((_RB, 1), lambda i: (i, 0)),
            pl.BlockSpec((_D, 1), lambda i: (0, 0)),
            pl.BlockSpec((1, 1), lambda i: (0, 0)),
            pl.BlockSpec((_D, 1), lambda i: (0, 0)),
            pl.BlockSpec((1, 1), lambda i: (0, 0)),
        ],
        out_specs=[
            pl.BlockSpec((_G, 1), lambda i: (0, 0)),
            pl.BlockSpec((_G, 1), lambda i: (0, 0)),
        ],
        out_shape=[
            jax.ShapeDtypeStruct((_G, 1), jnp.float32),
            jax.ShapeDtypeStruct((_G, 1), jnp.float32),
        ],
        scratch_shapes=[
            pltpu.VMEM((_G, _D), jnp.float32),
            pltpu.VMEM((_G, 1), jnp.float32),
        ],
    )(sa, sb, ph, dinv, b2, batch2, Wm, bm, Wt, bt)


def kernel(x, edge_index, batch, W1, b1, W2, b2, Wm, bm, Wt, bt):
    src3 = edge_index[0].reshape(_NT * _NSUP, _NSCH, _CH)
    dst3 = edge_index[1].reshape(_NT * _NSUP, _NSCH, _CH)
    batch2 = batch.reshape(_N, 1)
    b1r = b1.reshape(1, _D)
    b2r = b2.reshape(1, _D)
    bmr = bm.reshape(1, 1)
    btr = bt.reshape(1, 1)

    degj = jax.ops.segment_sum(jnp.ones((_E,), jnp.float32), edge_index[1],
                               num_segments=_N)
    dega = jnp.broadcast_to(degj[:, None], (_N, 16))
    degb = jnp.zeros((_N, 16), jnp.float32)

    def _jax_agg(ph):
        s = jax.ops.segment_sum(ph[edge_index[0]], edge_index[1],
                                num_segments=_N)
        return jnp.concatenate([s, jnp.zeros_like(s)], axis=0)

    p1h, dinv = _tc1(x, W1, dega, degb)
    s1 = _jax_agg(p1h)
    p2h = _tc2(s1[:_N], s1[_N:], p1h, dinv, b1r, W2)
    s2 = _jax_agg(p2h)
    mem, time = _tc3(s2[:_N], s2[_N:], p2h, dinv, b2r, batch2,
                     Wm, bmr, Wt, btr)
    return mem.reshape(_G), time.reshape(_G)


# speedup vs baseline: 18.5676x; 6.0437x over previous
import functools

import jax
import jax.numpy as jnp
from jax import lax
from jax.experimental import pallas as pl
from jax.experimental.pallas import tpu as pltpu
from jax.experimental.pallas import tpu_sc as plsc

_N = 10000
_E = 320000
_D = 128
_G = 64
_NC = 2
_NS = 16
_NT = _NC * _NS
_EPT = _E // _NT
_CH = 80
_NCHK = _EPT // _CH
_NSUP = 5
_NSCH = _NCHK // _NSUP
_RB = 2000
_NRB = _N // _RB


def _mesh():
    return plsc.VectorSubcoreMesh(core_axis_name="c", subcore_axis_name="s")


def _sc_degree(dst3):
    @functools.partial(
        pl.kernel,
        out_type=jax.ShapeDtypeStruct((_NC * _N, 16), jnp.float32),
        mesh=_mesh(),
        scratch_types=[
            pltpu.VMEM((_NSCH * _CH,), jnp.int32),
            pltpu.VMEM((_CH,), jnp.int32),
            pltpu.VMEM((_CH, 16), jnp.float32),
            pltpu.VMEM((_CH, 16), jnp.float32),
            pltpu.VMEM_SHARED((_N, 16), jnp.float32),
        ],
    )
    def k(dst_hbm, out_hbm, idx_v, idx_c, ones_v, zbuf, acc):
        c = lax.axis_index("c")
        s = lax.axis_index("s")
        tid = c * _NS + s

        def fill(r, _):
            ones_v[r, :] = jnp.ones((16,), jnp.float32)
            zbuf[r, :] = jnp.zeros((16,), jnp.float32)
            return 0

        lax.fori_loop(0, _CH, fill, 0)

        def iota_idx(base):
            for j in range(_CH // 16):
                idx_c[pl.ds(j * 16, 16)] = (
                    lax.broadcasted_iota(jnp.int32, (16,), 0) + base + j * 16)

        def zero_chunk(r, _):
            iota_idx(s * 640 + r * 80)
            pltpu.sync_copy(zbuf, acc.at[idx_c])
            return 0

        nz = jnp.where(s < _NS - 1, 8, 5)
        lax.fori_loop(0, nz, zero_chunk, 0)

        plsc.subcore_barrier()

        def inner(g, _):
            for j in range(_CH // 16):
                idx_c[pl.ds(j * 16, 16)] = idx_v[pl.ds(g * _CH + j * 16, 16)]
            pltpu.sync_copy(ones_v, acc.at[idx_c], add=True)
            return 0

        def outer(u, _):
            pltpu.sync_copy(
                dst_hbm.at[pl.ds((tid * _NSUP + u) * _NSCH * _CH,
                                 _NSCH * _CH)], idx_v)
            lax.fori_loop(0, _NSCH, inner, 0)
            return 0

        lax.fori_loop(0, _NSUP, outer, 0)
        plsc.subcore_barrier()

        def rb_chunk(r, _):
            base = s * 640 + r * 80
            iota_idx(base)
            pltpu.sync_copy(acc.at[idx_c], zbuf)
            pltpu.sync_copy(zbuf, out_hbm.at[pl.ds(c * _N + base, _CH)])
            return 0

        lax.fori_loop(0, nz, rb_chunk, 0)

    return k(dst3)


def _sc_aggregate(ph, src3, dst3):
    @functools.partial(
        pl.kernel,
        out_type=jax.ShapeDtypeStruct((_NC * _N, _D), jnp.float32),
        mesh=_mesh(),
        scratch_types=[
            pltpu.VMEM((_NSCH * _CH,), jnp.int32),
            pltpu.VMEM((_NSCH * _CH,), jnp.int32),
            pltpu.VMEM((_CH,), jnp.int32),
            pltpu.VMEM((_CH,), jnp.int32),
            pltpu.VMEM((_CH, _D), jnp.float32),
            pltpu.VMEM((_CH, _D), jnp.float32),
            pltpu.VMEM_SHARED((_N, _D), jnp.float32),
            pltpu.SemaphoreType.DMA,
        ],
    )
    def k(ph_hbm, src_hbm, dst_hbm, out_hbm, src_v, dst_v, idx_s, idx_c,
          rows_v, stage_v, acc, sem):
        c = lax.axis_index("c")
        s = lax.axis_index("s")
        tid = c * _NS + s

        def fill_zero(r, _):
            for j in range(_D // 16):
                stage_v[r, pl.ds(j * 16, 16)] = jnp.zeros((16,), jnp.float32)
            return 0

        lax.fori_loop(0, _CH, fill_zero, 0)

        def zero_chunk(r, _):
            base = s * 640 + r * 80
            for j in range(_CH // 16):
                idx_c[pl.ds(j * 16, 16)] = (
                    lax.broadcasted_iota(jnp.int32, (16,), 0) + base + j * 16)
            pltpu.sync_copy(stage_v, acc.at[idx_c])
            return 0

        @pl.when(s < _NS - 1)
        def _():
            lax.fori_loop(0, 8, zero_chunk, 0)

        @pl.when(s == _NS - 1)
        def _():
            lax.fori_loop(0, 5, zero_chunk, 0)

        plsc.subcore_barrier()

        def inner(g, _):
            for j in range(_CH // 16):
                idx_s[pl.ds(j * 16, 16)] = src_v[pl.ds(g * _CH + j * 16, 16)]
                idx_c[pl.ds(j * 16, 16)] = dst_v[pl.ds(g * _CH + j * 16, 16)]
            pltpu.async_copy(ph_hbm.at[idx_s], rows_v, sem).wait()
            pltpu.sync_copy(rows_v, acc.at[idx_c], add=True)
            return 0

        def outer(u, _):
            base = (tid * _NSUP + u) * _NSCH * _CH
            pltpu.sync_copy(src_hbm.at[pl.ds(base, _NSCH * _CH)], src_v)
            pltpu.sync_copy(dst_hbm.at[pl.ds(base, _NSCH * _CH)], dst_v)
            lax.fori_loop(0, _NSCH, inner, 0)
            return 0

        lax.fori_loop(0, _NSUP, outer, 0)
        plsc.subcore_barrier()

        def rb_chunk(r, _):
            base = s * 640 + r * 80
            for j in range(_CH // 16):
                idx_c[pl.ds(j * 16, 16)] = (
                    lax.broadcasted_iota(jnp.int32, (16,), 0) + base + j * 16)
            pltpu.sync_copy(acc.at[idx_c], stage_v)
            pltpu.sync_copy(stage_v, out_hbm.at[pl.ds(c * _N + base, _CH)])
            return 0

        lax.fori_loop(0, jnp.where(s < _NS - 1, 8, 5), rb_chunk, 0)

    return k(ph, src3, dst3)


def _tc1_body(x_ref, w_ref, da_ref, db_ref, ph_ref, dinv_ref):
    deg = da_ref[:, 0:1] + db_ref[:, 0:1] + 1.0
    dinv = lax.rsqrt(deg)
    dinv_ref[...] = dinv
    ph_ref[...] = jnp.dot(x_ref[...], w_ref[...],
                          preferred_element_type=jnp.float32) * dinv


def _tc1(x, W1, dega, degb):
    return pl.pallas_call(
        _tc1_body,
        grid=(_NRB,),
        in_specs=[
            pl.BlockSpec((_RB, _D), lambda i: (i, 0)),
            pl.BlockSpec((_D, _D), lambda i: (0, 0)),
            pl.BlockSpec((_RB, 16), lambda i: (i, 0)),
            pl.BlockSpec((_RB, 16), lambda i: (i, 0)),
        ],
        out_specs=[
            pl.BlockSpec((_RB, _D), lambda i: (i, 0)),
            pl.BlockSpec((_RB, 1), lambda i: (i, 0)),
        ],
        out_shape=[
            jax.ShapeDtypeStruct((_N, _D), jnp.float32),
            jax.ShapeDtypeStruct((_N, 1), jnp.float32),
        ],
    )(x, W1, dega, degb)


def _tc2_body(sa_ref, sb_ref, ph_ref, dinv_ref, b_ref, w_ref, out_ref):
    dinv = dinv_ref[...]
    t = sa_ref[...] + sb_ref[...] + ph_ref[...]
    h = jnp.maximum(t * dinv + b_ref[...], 0.0)
    out_ref[...] = jnp.dot(h, w_ref[...],
                           preferred_element_type=jnp.float32) * dinv


def _tc2(sa, sb, ph, dinv, b1, W2):
    return pl.pallas_call(
        _tc2_body,
        grid=(_NRB,),
        in_specs=[
            pl.BlockSpec((_RB, _D), lambda i: (i, 0)),
            pl.BlockSpec((_RB, _D), lambda i: (i, 0)),
            pl.BlockSpec((_RB, _D), lambda i: (i, 0)),
            pl.BlockSpec((_RB, 1), lambda i: (i, 0)),
            pl.BlockSpec((1, _D), lambda i: (0, 0)),
            pl.BlockSpec((_D, _D), lambda i: (0, 0)),
        ],
        out_specs=pl.BlockSpec((_RB, _D), lambda i: (i, 0)),
        out_shape=jax.ShapeDtypeStruct((_N, _D), jnp.float32),
    )(sa, sb, ph, dinv, b1, W2)


def _tc3_body(sa_ref, sb_ref, ph_ref, dinv_ref, b_ref, batch_ref,
              wm_ref, bm_ref, wt_ref, bt_ref, mem_ref, time_ref,
              pool_ref, cnt_ref):
    i = pl.program_id(0)
    t = sa_ref[...] + sb_ref[...] + ph_ref[...]
    h = jnp.maximum(t * dinv_ref[...] + b_ref[...], 0.0)
    gids = lax.broadcasted_iota(jnp.int32, (1, _G), 1)
    onehot = (batch_ref[...] == gids).astype(jnp.float32)
    dn = (((0,), (0,)), ((), ()))
    pp = lax.dot_general(onehot, h, dn,
                         preferred_element_type=jnp.float32)
    cc = lax.dot_general(onehot, jnp.ones((_RB, 1), jnp.float32), dn,
                         preferred_element_type=jnp.float32)

    @pl.when(i == 0)
    def _():
        pool_ref[...] = pp
        cnt_ref[...] = cc

    @pl.when(i > 0)
    def _():
        pool_ref[...] += pp
        cnt_ref[...] += cc

    @pl.when(i == _NRB - 1)
    def _():
        mean = pool_ref[...] / jnp.maximum(cnt_ref[...], 1.0)
        mem_ref[...] = jnp.dot(mean, wm_ref[...],
                               preferred_element_type=jnp.float32) + bm_ref[...]
        time_ref[...] = jnp.dot(mean, wt_ref[...],
                                preferred_element_type=jnp.float32) + bt_ref[...]


def _tc3(sa, sb, ph, dinv, b2, batch2, Wm, bm, Wt, bt):
    return pl.pallas_call(
        _tc3_body,
        grid=(_NRB,),
        in_specs=[
            pl.BlockSpec((_RB, _D), lambda i: (i, 0)),
            pl.BlockSpec((_RB, _D), lambda i: (i, 0)),
            pl.BlockSpec((_RB, _D), lambda i: (i, 0)),
            pl.BlockSpec((_RB, 1), lambda i: (i, 0)),
            pl.BlockSpec((1, _D), lambda i: (0, 0)),
            pl.BlockSpec((_RB, 1), lambda i: (i, 0)),
            pl.BlockSpec((_D, 1), lambda i: (0, 0)),
            pl.BlockSpec((1, 1), lambda i: (0, 0)),
            pl.BlockSpec((_D, 1), lambda i: (0, 0)),
            pl.BlockSpec((1, 1), lambda i: (0, 0)),
        ],
        out_specs=[
            pl.BlockSpec((_G, 1), lambda i: (0, 0)),
            pl.BlockSpec((_G, 1), lambda i: (0, 0)),
        ],
        out_shape=[
            jax.ShapeDtypeStruct((_G, 1), jnp.float32),
            jax.ShapeDtypeStruct((_G, 1), jnp.float32),
        ],
        scratch_shapes=[
            pltpu.VMEM((_G, _D), jnp.float32),
            pltpu.VMEM((_G, 1), jnp.float32),
        ],
    )(sa, sb, ph, dinv, b2, batch2, Wm, bm, Wt, bt)


def kernel(x, edge_index, batch, W1, b1, W2, b2, Wm, bm, Wt, bt):
    src3 = edge_index[0]
    dst3 = edge_index[1]
    batch2 = batch.reshape(_N, 1)
    b1r = b1.reshape(1, _D)
    b2r = b2.reshape(1, _D)
    bmr = bm.reshape(1, 1)
    btr = bt.reshape(1, 1)

    deg = _sc_degree(dst3)
    dega, degb = deg[:_N], deg[_N:]

    p1h, dinv = _tc1(x, W1, dega, degb)
    s1 = _sc_aggregate(p1h, src3, dst3)
    p2h = _tc2(s1[:_N], s1[_N:], p1h, dinv, b1r, W2)
    s2 = _sc_aggregate(p2h, src3, dst3)
    mem, time = _tc3(s2[:_N], s2[_N:], p2h, dinv, b2r, batch2,
                     Wm, bmr, Wt, btr)
    return mem.reshape(_G), time.reshape(_G)


# speedup vs baseline: 22.7011x; 1.2226x over previous
import functools

import jax
import jax.numpy as jnp
from jax import lax
from jax.experimental import pallas as pl
from jax.experimental.pallas import tpu as pltpu
from jax.experimental.pallas import tpu_sc as plsc

_N = 10000
_E = 320000
_D = 128
_G = 64
_NC = 2
_NS = 16
_NT = _NC * _NS
_EPT = _E // _NT
_CH = 80
_NCHK = _EPT // _CH
_NSUP = 5
_NSCH = _NCHK // _NSUP
_RB = 2000
_NRB = _N // _RB


def _mesh():
    return plsc.VectorSubcoreMesh(core_axis_name="c", subcore_axis_name="s")


def _sc_degree(dst3):
    @functools.partial(
        pl.kernel,
        out_type=jax.ShapeDtypeStruct((_NC * _N, 16), jnp.float32),
        mesh=_mesh(),
        scratch_types=[
            pltpu.VMEM((_NSCH * _CH,), jnp.int32),
            pltpu.VMEM((_CH,), jnp.int32),
            pltpu.VMEM((_CH, 16), jnp.float32),
            pltpu.VMEM((_CH, 16), jnp.float32),
            pltpu.VMEM_SHARED((_N, 16), jnp.float32),
        ],
    )
    def k(dst_hbm, out_hbm, idx_v, idx_c, ones_v, zbuf, acc):
        c = lax.axis_index("c")
        s = lax.axis_index("s")
        tid = c * _NS + s

        def fill(r, _):
            ones_v[r, :] = jnp.ones((16,), jnp.float32)
            zbuf[r, :] = jnp.zeros((16,), jnp.float32)
            return 0

        lax.fori_loop(0, _CH, fill, 0)

        def iota_idx(base):
            for j in range(_CH // 16):
                idx_c[pl.ds(j * 16, 16)] = (
                    lax.broadcasted_iota(jnp.int32, (16,), 0) + base + j * 16)

        def zero_chunk(r, _):
            iota_idx(s * 640 + r * 80)
            pltpu.sync_copy(zbuf, acc.at[idx_c])
            return 0

        nz = jnp.where(s < _NS - 1, 8, 5)
        lax.fori_loop(0, nz, zero_chunk, 0)

        plsc.subcore_barrier()

        def inner(g, _):
            for j in range(_CH // 16):
                idx_c[pl.ds(j * 16, 16)] = idx_v[pl.ds(g * _CH + j * 16, 16)]
            pltpu.sync_copy(ones_v, acc.at[idx_c], add=True)
            return 0

        def outer(u, _):
            pltpu.sync_copy(
                dst_hbm.at[pl.ds((tid * _NSUP + u) * _NSCH * _CH,
                                 _NSCH * _CH)], idx_v)
            lax.fori_loop(0, _NSCH, inner, 0)
            return 0

        lax.fori_loop(0, _NSUP, outer, 0)
        plsc.subcore_barrier()

        def rb_chunk(r, _):
            base = s * 640 + r * 80
            iota_idx(base)
            pltpu.sync_copy(acc.at[idx_c], zbuf)
            pltpu.sync_copy(zbuf, out_hbm.at[pl.ds(c * _N + base, _CH)])
            return 0

        lax.fori_loop(0, nz, rb_chunk, 0)

    return k(dst3)


def _sc_aggregate(ph, src3, dst3):
    @functools.partial(
        pl.kernel,
        out_type=jax.ShapeDtypeStruct((_NC * _N, _D), jnp.float32),
        mesh=_mesh(),
        scratch_types=[
            pltpu.VMEM((_NSCH * _CH,), jnp.int32),
            pltpu.VMEM((_NSCH * _CH,), jnp.int32),
            pltpu.VMEM((_CH,), jnp.int32),
            pltpu.VMEM((_CH,), jnp.int32),
            pltpu.VMEM((_CH,), jnp.int32),
            pltpu.VMEM((_CH, _D), jnp.float32),
            pltpu.VMEM((_CH, _D), jnp.float32),
            pltpu.VMEM((_CH, _D), jnp.float32),
            pltpu.VMEM_SHARED((_N, _D), jnp.float32),
            pltpu.SemaphoreType.DMA,
        ],
    )
    def k(ph_hbm, src_hbm, dst_hbm, out_hbm, src_v, dst_v, idx_sa, idx_sb,
          idx_c, rows_a, rows_b, stage_v, acc, sem):
        c = lax.axis_index("c")
        s = lax.axis_index("s")
        tid = c * _NS + s

        def fill_zero(r, _):
            for j in range(_D // 16):
                stage_v[r, pl.ds(j * 16, 16)] = jnp.zeros((16,), jnp.float32)
            return 0

        lax.fori_loop(0, _CH, fill_zero, 0)

        def zero_chunk(r, _):
            base = s * 640 + r * 80
            for j in range(_CH // 16):
                idx_c[pl.ds(j * 16, 16)] = (
                    lax.broadcasted_iota(jnp.int32, (16,), 0) + base + j * 16)
            pltpu.sync_copy(stage_v, acc.at[idx_c])
            return 0

        @pl.when(s < _NS - 1)
        def _():
            lax.fori_loop(0, 8, zero_chunk, 0)

        @pl.when(s == _NS - 1)
        def _():
            lax.fori_loop(0, 5, zero_chunk, 0)

        plsc.subcore_barrier()

        def cp_src(g, dst_ref):
            for j in range(_CH // 16):
                dst_ref[pl.ds(j * 16, 16)] = (
                    src_v[pl.ds(g * _CH + j * 16, 16)])

        def inner(g, _):
            even = (g % 2) == 0

            @pl.when(even)
            def _():
                pltpu.make_async_copy(ph_hbm.at[idx_sa], rows_a, sem).wait()

            @pl.when(jnp.logical_not(even))
            def _():
                pltpu.make_async_copy(ph_hbm.at[idx_sb], rows_b, sem).wait()

            @pl.when(even & (g < _NSCH - 1))
            def _():
                cp_src(g + 1, idx_sb)
                pltpu.async_copy(ph_hbm.at[idx_sb], rows_b, sem)

            @pl.when(jnp.logical_not(even) & (g < _NSCH - 1))
            def _():
                cp_src(g + 1, idx_sa)
                pltpu.async_copy(ph_hbm.at[idx_sa], rows_a, sem)

            for j in range(_CH // 16):
                idx_c[pl.ds(j * 16, 16)] = dst_v[pl.ds(g * _CH + j * 16, 16)]

            @pl.when(even)
            def _():
                pltpu.sync_copy(rows_a, acc.at[idx_c], add=True)

            @pl.when(jnp.logical_not(even))
            def _():
                pltpu.sync_copy(rows_b, acc.at[idx_c], add=True)

            return 0

        def outer(u, _):
            base = (tid * _NSUP + u) * _NSCH * _CH
            pltpu.sync_copy(src_hbm.at[pl.ds(base, _NSCH * _CH)], src_v)
            pltpu.sync_copy(dst_hbm.at[pl.ds(base, _NSCH * _CH)], dst_v)
            cp_src(0, idx_sa)
            pltpu.async_copy(ph_hbm.at[idx_sa], rows_a, sem)
            lax.fori_loop(0, _NSCH, inner, 0)
            return 0

        lax.fori_loop(0, _NSUP, outer, 0)
        plsc.subcore_barrier()

        def rb_chunk(r, _):
            base = s * 640 + r * 80
            for j in range(_CH // 16):
                idx_c[pl.ds(j * 16, 16)] = (
                    lax.broadcasted_iota(jnp.int32, (16,), 0) + base + j * 16)
            pltpu.sync_copy(acc.at[idx_c], stage_v)
            pltpu.sync_copy(stage_v, out_hbm.at[pl.ds(c * _N + base, _CH)])
            return 0

        lax.fori_loop(0, jnp.where(s < _NS - 1, 8, 5), rb_chunk, 0)

    return k(ph, src3, dst3)


def _tc1_body(x_ref, w_ref, da_ref, db_ref, ph_ref, dinv_ref):
    deg = da_ref[:, 0:1] + db_ref[:, 0:1] + 1.0
    dinv = lax.rsqrt(deg)
    dinv_ref[...] = dinv
    ph_ref[...] = jnp.dot(x_ref[...], w_ref[...],
                          preferred_element_type=jnp.float32) * dinv


def _tc1(x, W1, dega, degb):
    return pl.pallas_call(
        _tc1_body,
        grid=(_NRB,),
        in_specs=[
            pl.BlockSpec((_RB, _D), lambda i: (i, 0)),
            pl.BlockSpec((_D, _D), lambda i: (0, 0)),
            pl.BlockSpec((_RB, 16), lambda i: (i, 0)),
            pl.BlockSpec((_RB, 16), lambda i: (i, 0)),
        ],
        out_specs=[
            pl.BlockSpec((_RB, _D), lambda i: (i, 0)),
            pl.BlockSpec((_RB, 1), lambda i: (i, 0)),
        ],
        out_shape=[
            jax.ShapeDtypeStruct((_N, _D), jnp.float32),
            jax.ShapeDtypeStruct((_N, 1), jnp.float32),
        ],
    )(x, W1, dega, degb)


def _tc2_body(sa_ref, sb_ref, ph_ref, dinv_ref, b_ref, w_ref, out_ref):
    dinv = dinv_ref[...]
    t = sa_ref[...] + sb_ref[...] + ph_ref[...]
    h = jnp.maximum(t * dinv + b_ref[...], 0.0)
    out_ref[...] = jnp.dot(h, w_ref[...],
                           preferred_element_type=jnp.float32) * dinv


def _tc2(sa, sb, ph, dinv, b1, W2):
    return pl.pallas_call(
        _tc2_body,
        grid=(_NRB,),
        in_specs=[
            pl.BlockSpec((_RB, _D), lambda i: (i, 0)),
            pl.BlockSpec((_RB, _D), lambda i: (i, 0)),
            pl.BlockSpec((_RB, _D), lambda i: (i, 0)),
            pl.BlockSpec((_RB, 1), lambda i: (i, 0)),
            pl.BlockSpec((1, _D), lambda i: (0, 0)),
            pl.BlockSpec((_D, _D), lambda i: (0, 0)),
        ],
        out_specs=pl.BlockSpec((_RB, _D), lambda i: (i, 0)),
        out_shape=jax.ShapeDtypeStruct((_N, _D), jnp.float32),
    )(sa, sb, ph, dinv, b1, W2)


def _tc3_body(sa_ref, sb_ref, ph_ref, dinv_ref, b_ref, batch_ref,
              wm_ref, bm_ref, wt_ref, bt_ref, mem_ref, time_ref,
              pool_ref, cnt_ref):
    i = pl.program_id(0)
    t = sa_ref[...] + sb_ref[...] + ph_ref[...]
    h = jnp.maximum(t * dinv_ref[...] + b_ref[...], 0.0)
    gids = lax.broadcasted_iota(jnp.int32, (1, _G), 1)
    onehot = (batch_ref[...] == gids).astype(jnp.float32)
    dn = (((0,), (0,)), ((), ()))
    pp = lax.dot_general(onehot, h, dn,
                         preferred_element_type=jnp.float32)
    cc = lax.dot_general(onehot, jnp.ones((_RB, 1), jnp.float32), dn,
                         preferred_element_type=jnp.float32)

    @pl.when(i == 0)
    def _():
        pool_ref[...] = pp
        cnt_ref[...] = cc

    @pl.when(i > 0)
    def _():
        pool_ref[...] += pp
        cnt_ref[...] += cc

    @pl.when(i == _NRB - 1)
    def _():
        mean = pool_ref[...] / jnp.maximum(cnt_ref[...], 1.0)
        mem_ref[...] = jnp.dot(mean, wm_ref[...],
                               preferred_element_type=jnp.float32) + bm_ref[...]
        time_ref[...] = jnp.dot(mean, wt_ref[...],
                                preferred_element_type=jnp.float32) + bt_ref[...]


def _tc3(sa, sb, ph, dinv, b2, batch2, Wm, bm, Wt, bt):
    return pl.pallas_call(
        _tc3_body,
        grid=(_NRB,),
        in_specs=[
            pl.BlockSpec((_RB, _D), lambda i: (i, 0)),
            pl.BlockSpec((_RB, _D), lambda i: (i, 0)),
            pl.BlockSpec((_RB, _D), lambda i: (i, 0)),
            pl.BlockSpec((_RB, 1), lambda i: (i, 0)),
            pl.BlockSpec((1, _D), lambda i: (0, 0)),
            pl.BlockSpec((_RB, 1), lambda i: (i, 0)),
            pl.BlockSpec((_D, 1), lambda i: (0, 0)),
            pl.BlockSpec((1, 1), lambda i: (0, 0)),
            pl.BlockSpec((_D, 1), lambda i: (0, 0)),
            pl.BlockSpec((1, 1), lambda i: (0, 0)),
        ],
        out_specs=[
            pl.BlockSpec((_G, 1), lambda i: (0, 0)),
            pl.BlockSpec((_G, 1), lambda i: (0, 0)),
        ],
        out_shape=[
            jax.ShapeDtypeStruct((_G, 1), jnp.float32),
            jax.ShapeDtypeStruct((_G, 1), jnp.float32),
        ],
        scratch_shapes=[
            pltpu.VMEM((_G, _D), jnp.float32),
            pltpu.VMEM((_G, 1), jnp.float32),
        ],
    )(sa, sb, ph, dinv, b2, batch2, Wm, bm, Wt, bt)


def kernel(x, edge_index, batch, W1, b1, W2, b2, Wm, bm, Wt, bt):
    src3 = edge_index[0]
    dst3 = edge_index[1]
    batch2 = batch.reshape(_N, 1)
    b1r = b1.reshape(1, _D)
    b2r = b2.reshape(1, _D)
    bmr = bm.reshape(1, 1)
    btr = bt.reshape(1, 1)

    deg = _sc_degree(dst3)
    dega, degb = deg[:_N], deg[_N:]

    p1h, dinv = _tc1(x, W1, dega, degb)
    s1 = _sc_aggregate(p1h, src3, dst3)
    p2h = _tc2(s1[:_N], s1[_N:], p1h, dinv, b1r, W2)
    s2 = _sc_aggregate(p2h, src3, dst3)
    mem, time = _tc3(s2[:_N], s2[_N:], p2h, dinv, b2r, batch2,
                     Wm, bmr, Wt, btr)
    return mem.reshape(_G), time.reshape(_G)
